# baseline (speedup 1.0000x reference)
import functools

import jax
import jax.numpy as jnp
import numpy as np
from jax import lax
from jax.experimental import pallas as pl
from jax.experimental.pallas import tpu as pltpu
from jax.experimental.pallas import tpu_sc as plsc

N_PAIRS = 65536
BATCH = 16384
DIM = 64
NC = 2
NS = 16
NW = NC * NS
PAIRS_PER_W = N_PAIRS // NW
CHUNK = 64
N_CHUNKS = PAIRS_PER_W // CHUNK
LANES = 16
GROUPS = CHUNK // LANES
DEPTH = 4

INV3 = np.uint32(0xAAAAAAAB)
THRESH3 = np.uint32(0x55555555)


def _padic_d3(bi, bj):
    diff = jnp.abs(bi - bj).astype(jnp.uint32)
    cur = diff
    v = jnp.zeros((LANES,), jnp.int32)
    for _ in range(13):
        m = cur * INV3
        div3 = m <= THRESH3
        v = v + div3.astype(jnp.int32)
        cur = jnp.where(div3, m, cur)
    r = jnp.full((LANES,), 1.0, jnp.float32)
    t = 3.0
    for bit in range(4):
        r = jnp.where(((v >> bit) & 1) != 0, r * t, r)
        t = t * t
    d3 = 1.0 / r
    return jnp.where(diff == 0, jnp.zeros((LANES,), jnp.float32), d3)


def _sqrt16(x):
    xi = lax.bitcast_convert_type(x, jnp.int32)
    yi = jnp.int32(0x5F3759DF) - (xi >> 1)
    y = lax.bitcast_convert_type(yi, jnp.float32)
    for _ in range(3):
        y = y * (1.5 - 0.5 * x * y * y)
    return jnp.where(x > 0, x * y, jnp.zeros((LANES,), jnp.float32))


def _splat_i32(x):
    return jnp.zeros((LANES,), jnp.int32) + x


def _body(z_hbm, b_hbm, i_hbm, j_hbm, out_hbm,
          i_all, j_all, zi0, zj0, zi1, zj1, zi2, zj2, zi3, zj3,
          b_all, acc_v, z_sp,
          si0, sj0, si1, sj1, si2, sj2, si3, sj3, sb):
    sid = lax.axis_index("s")
    wid = sid * NC + lax.axis_index("c")
    lane = lax.iota(jnp.int32, LANES)

    rows_per_sid = BATCH // NS
    cz = pltpu.async_copy(
        z_hbm.at[pl.ds(sid * rows_per_sid, rows_per_sid)],
        z_sp.at[pl.ds(sid * rows_per_sid, rows_per_sid)], sb)
    cb = pltpu.async_copy(b_hbm, b_all, sb)
    ci = pltpu.async_copy(i_hbm.at[pl.ds(wid * N_CHUNKS, N_CHUNKS)], i_all, si0)
    cj = pltpu.async_copy(j_hbm.at[pl.ds(wid * N_CHUNKS, N_CHUNKS)], j_all, sj0)
    ci.wait()
    cj.wait()

    def fix(g, carry):
        row = _splat_i32(g >> 2)
        col = (g & 3) * LANES + lane
        iv = plsc.load_gather(i_all, [row, col])
        jv = plsc.load_gather(j_all, [row, col])
        jf = jnp.where(iv == jv, (jv + 1) & (BATCH - 1), jv)
        plsc.store_scatter(j_all, [row, col], jf)
        return carry

    lax.fori_loop(0, N_CHUNKS * GROUPS, fix, 0)
    cz.wait()
    cb.wait()
    plsc.subcore_barrier()

    def issue(c, zi_v, zj_v, sem_i, sem_j):
        pltpu.async_copy(z_sp.at[i_all.at[c]], zi_v, sem_i)
        pltpu.async_copy(z_sp.at[j_all.at[c]], zj_v, sem_j)

    def wait(c, zi_v, zj_v, sem_i, sem_j):
        pltpu.make_async_copy(z_sp.at[i_all.at[c]], zi_v, sem_i).wait()
        pltpu.make_async_copy(z_sp.at[j_all.at[c]], zj_v, sem_j).wait()

    def compute(c, zi_v, zj_v, acc):
        def group(g, acc):
            p = g * LANES + lane
            row = _splat_i32(c)
            iv = plsc.load_gather(i_all, [row, p])
            jv = plsc.load_gather(j_all, [row, p])
            d2 = jnp.zeros((LANES,), jnp.float32)
            for d in range(DIM):
                col = (lane + d) & (DIM - 1)
                a = plsc.load_gather(zi_v, [p, col])
                b = plsc.load_gather(zj_v, [p, col])
                df = a - b
                d2 = d2 + df * df
            d_lat = _sqrt16(d2)
            bi = plsc.load_gather(b_all, [iv])
            bj = plsc.load_gather(b_all, [jv])
            err = d_lat - _padic_d3(bi, bj)
            return acc + err * err

        return lax.fori_loop(0, GROUPS, group, acc)

    bufs = ((zi0, zj0, si0, sj0), (zi1, zj1, si1, sj1),
            (zi2, zj2, si2, sj2), (zi3, zj3, si3, sj3))
    for k in range(DEPTH):
        issue(k, *bufs[k])

    def ring(it, acc):
        for k in range(DEPTH):
            c = DEPTH * it + k
            wait(c, *bufs[k])
            acc = compute(c, bufs[k][0], bufs[k][1], acc)

            @pl.when(it < N_CHUNKS // DEPTH - 1)
            def _():
                issue(c + DEPTH, *bufs[k])

        return acc

    acc = lax.fori_loop(0, N_CHUNKS // DEPTH, ring,
                        jnp.zeros((LANES,), jnp.float32))

    acc_v[...] = acc * (1.0 / N_PAIRS)
    pltpu.sync_copy(acc_v, out_hbm.at[pl.ds(wid * LANES, LANES)])


_sc_call = functools.partial(
    pl.kernel,
    out_type=jax.ShapeDtypeStruct((NW * LANES,), jnp.float32),
    mesh=plsc.VectorSubcoreMesh(core_axis_name="c", subcore_axis_name="s"),
    compiler_params=pltpu.CompilerParams(
        needs_layout_passes=False, use_tc_tiling_on_sc=False),
    scratch_types=[
        pltpu.VMEM((N_CHUNKS, CHUNK), jnp.int32),
        pltpu.VMEM((N_CHUNKS, CHUNK), jnp.int32),
        pltpu.VMEM((CHUNK, DIM), jnp.float32),
        pltpu.VMEM((CHUNK, DIM), jnp.float32),
        pltpu.VMEM((CHUNK, DIM), jnp.float32),
        pltpu.VMEM((CHUNK, DIM), jnp.float32),
        pltpu.VMEM((CHUNK, DIM), jnp.float32),
        pltpu.VMEM((CHUNK, DIM), jnp.float32),
        pltpu.VMEM((CHUNK, DIM), jnp.float32),
        pltpu.VMEM((CHUNK, DIM), jnp.float32),
        pltpu.VMEM((BATCH,), jnp.int32),
        pltpu.VMEM((LANES,), jnp.float32),
        pltpu.VMEM_SHARED((BATCH, DIM), jnp.float32),
        pltpu.SemaphoreType.DMA,
        pltpu.SemaphoreType.DMA,
        pltpu.SemaphoreType.DMA,
        pltpu.SemaphoreType.DMA,
        pltpu.SemaphoreType.DMA,
        pltpu.SemaphoreType.DMA,
        pltpu.SemaphoreType.DMA,
        pltpu.SemaphoreType.DMA,
        pltpu.SemaphoreType.DMA,
    ],
)(_body)


@jax.jit
def kernel(z, batch_indices, i_idx, j_idx):
    i2 = i_idx.reshape(N_PAIRS // CHUNK, CHUNK)
    j2 = j_idx.reshape(N_PAIRS // CHUNK, CHUNK)
    partials = _sc_call(z, batch_indices, i2, j2)
    return jnp.sum(partials)


# speedup vs baseline: 1.2561x; 1.2561x over previous
import functools

import jax
import jax.numpy as jnp
import numpy as np
from jax import lax
from jax.experimental import pallas as pl
from jax.experimental.pallas import tpu as pltpu
from jax.experimental.pallas import tpu_sc as plsc

N_PAIRS = 65536
BATCH = 16384
DIM = 64
NC = 2
NS = 16
NW = NC * NS
PAIRS_PER_W = N_PAIRS // NW
CHUNK = 128
N_CHUNKS = PAIRS_PER_W // CHUNK
LANES = 16
GROUPS = CHUNK // LANES
DIMP = DIM // 2

INV3 = np.uint32(0xAAAAAAAB)
THRESH3 = np.uint32(0x55555555)


def _padic_d3(bi, bj):
    diff = jnp.abs(bi - bj).astype(jnp.uint32)
    cur = diff
    v = jnp.zeros((LANES,), jnp.int32)
    for _ in range(12):
        m = cur * INV3
        div3 = m <= THRESH3
        v = v + div3.astype(jnp.int32)
        cur = jnp.where(div3, m, cur)
    r = jnp.full((LANES,), 1.0, jnp.float32)
    t = 3.0
    for bit in range(4):
        r = jnp.where(((v >> bit) & 1) != 0, r * t, r)
        t = t * t
    d3 = 1.0 / r
    return jnp.where(diff == 0, jnp.zeros((LANES,), jnp.float32), d3)


def _sqrt16(x):
    xi = lax.bitcast_convert_type(x, jnp.int32)
    yi = jnp.int32(0x5F3759DF) - (xi >> 1)
    y = lax.bitcast_convert_type(yi, jnp.float32)
    for _ in range(2):
        y = y * (1.5 - 0.5 * x * y * y)
    return jnp.where(x > 0, x * y, jnp.zeros((LANES,), jnp.float32))


def _splat_i32(x):
    return jnp.zeros((LANES,), jnp.int32) + x


def _body(z_hbm, b_hbm, i_hbm, j_hbm, out_hbm,
          i_all, j_all, zi0, zj0, zi1, zj1,
          b_all, acc_v, z_sp,
          si0, sj0, si1, sj1, sb):
    sid = lax.axis_index("s")
    wid = sid * NC + lax.axis_index("c")
    lane = lax.iota(jnp.int32, LANES)

    rows_per_sid = BATCH // NS
    cz = pltpu.async_copy(
        z_hbm.at[pl.ds(sid * rows_per_sid, rows_per_sid)],
        z_sp.at[pl.ds(sid * rows_per_sid, rows_per_sid)], sb)
    cb = pltpu.async_copy(b_hbm, b_all, sb)
    ci = pltpu.async_copy(i_hbm.at[pl.ds(wid * N_CHUNKS, N_CHUNKS)], i_all, si0)
    cj = pltpu.async_copy(j_hbm.at[pl.ds(wid * N_CHUNKS, N_CHUNKS)], j_all, sj0)
    ci.wait()
    cj.wait()

    def fix(g, carry):
        row = _splat_i32(g >> 3)
        col = (g & 7) * LANES + lane
        iv = plsc.load_gather(i_all, [row, col])
        jv = plsc.load_gather(j_all, [row, col])
        jf = jnp.where(iv == jv, (jv + 1) & (BATCH - 1), jv)
        plsc.store_scatter(j_all, [row, col], jf)
        return carry

    lax.fori_loop(0, N_CHUNKS * GROUPS, fix, 0)
    cz.wait()
    cb.wait()
    plsc.subcore_barrier()

    def issue(c, zi_v, zj_v, sem_i, sem_j):
        pltpu.async_copy(z_sp.at[i_all.at[c]], zi_v, sem_i)
        pltpu.async_copy(z_sp.at[j_all.at[c]], zj_v, sem_j)

    def wait(c, zi_v, zj_v, sem_i, sem_j):
        pltpu.make_async_copy(z_sp.at[i_all.at[c]], zi_v, sem_i).wait()
        pltpu.make_async_copy(z_sp.at[j_all.at[c]], zj_v, sem_j).wait()

    def compute(c, zi_v, zj_v, acc):
        def group(g, acc):
            p = g * LANES + lane
            row = _splat_i32(c)
            iv = plsc.load_gather(i_all, [row, p])
            jv = plsc.load_gather(j_all, [row, p])
            d2 = jnp.zeros((LANES,), jnp.float32)
            for d in range(DIM):
                col = (lane + d) & (DIM - 1)
                a = plsc.load_gather(zi_v, [p, col])
                b = plsc.load_gather(zj_v, [p, col])
                df = a - b
                d2 = d2 + df * df
            d_lat = _sqrt16(d2)
            bi = plsc.load_gather(b_all, [iv])
            bj = plsc.load_gather(b_all, [jv])
            err = d_lat - _padic_d3(bi, bj)
            return acc + err * err

        return lax.fori_loop(0, GROUPS, group, acc)

    issue(0, zi0, zj0, si0, sj0)

    def ring(it, acc):
        c0 = 2 * it
        issue(c0 + 1, zi1, zj1, si1, sj1)
        wait(c0, zi0, zj0, si0, sj0)
        acc = compute(c0, zi0, zj0, acc)
        issue(c0 + 2, zi0, zj0, si0, sj0)
        wait(c0 + 1, zi1, zj1, si1, sj1)
        return compute(c0 + 1, zi1, zj1, acc)

    acc = lax.fori_loop(0, N_CHUNKS // 2 - 1, ring,
                        jnp.zeros((LANES,), jnp.float32))
    issue(N_CHUNKS - 1, zi1, zj1, si1, sj1)
    wait(N_CHUNKS - 2, zi0, zj0, si0, sj0)
    acc = compute(N_CHUNKS - 2, zi0, zj0, acc)
    wait(N_CHUNKS - 1, zi1, zj1, si1, sj1)
    acc = compute(N_CHUNKS - 1, zi1, zj1, acc)

    acc_v[...] = acc * (1.0 / N_PAIRS)
    pltpu.sync_copy(acc_v, out_hbm.at[pl.ds(wid * LANES, LANES)])


_sc_call = functools.partial(
    pl.kernel,
    out_type=jax.ShapeDtypeStruct((NW * LANES,), jnp.float32),
    mesh=plsc.VectorSubcoreMesh(core_axis_name="c", subcore_axis_name="s"),
    compiler_params=pltpu.CompilerParams(
        needs_layout_passes=False, use_tc_tiling_on_sc=False),
    scratch_types=[
        pltpu.VMEM((N_CHUNKS, CHUNK), jnp.int32),
        pltpu.VMEM((N_CHUNKS, CHUNK), jnp.int32),
        pltpu.VMEM((CHUNK, DIM), jnp.float32),
        pltpu.VMEM((CHUNK, DIM), jnp.float32),
        pltpu.VMEM((CHUNK, DIM), jnp.float32),
        pltpu.VMEM((CHUNK, DIM), jnp.float32),
        pltpu.VMEM((BATCH,), jnp.int32),
        pltpu.VMEM((LANES,), jnp.float32),
        pltpu.VMEM_SHARED((BATCH, DIM), jnp.float32),
        pltpu.SemaphoreType.DMA,
        pltpu.SemaphoreType.DMA,
        pltpu.SemaphoreType.DMA,
        pltpu.SemaphoreType.DMA,
        pltpu.SemaphoreType.DMA,
    ],
)(_body)


@jax.jit
def kernel(z, batch_indices, i_idx, j_idx):
    i2 = i_idx.reshape(N_PAIRS // CHUNK, CHUNK)
    j2 = j_idx.reshape(N_PAIRS // CHUNK, CHUNK)
    partials = _sc_call(z, batch_indices, i2, j2)
    return jnp.sum(partials)


# speedup vs baseline: 1.2579x; 1.0014x over previous
import functools

import jax
import jax.numpy as jnp
import numpy as np
from jax import lax
from jax.experimental import pallas as pl
from jax.experimental.pallas import tpu as pltpu
from jax.experimental.pallas import tpu_sc as plsc

N_PAIRS = 65536
BATCH = 16384
DIM = 64
NC = 2
NS = 16
NW = NC * NS
PAIRS_PER_W = N_PAIRS // NW
CHUNK = 128
N_CHUNKS = PAIRS_PER_W // CHUNK
LANES = 16
GROUPS = CHUNK // LANES

INV3 = np.uint32(0xAAAAAAAB)
THRESH3 = np.uint32(0x55555555)


def _padic_d3(bi, bj):
    diff = jnp.abs(bi - bj).astype(jnp.uint32)
    cur = diff
    v = jnp.zeros((LANES,), jnp.int32)
    for _ in range(12):
        m = cur * INV3
        div3 = m <= THRESH3
        v = v + div3.astype(jnp.int32)
        cur = jnp.where(div3, m, cur)
    r = jnp.full((LANES,), 1.0, jnp.float32)
    t = 3.0
    for bit in range(4):
        r = jnp.where(((v >> bit) & 1) != 0, r * t, r)
        t = t * t
    d3 = 1.0 / r
    return jnp.where(diff == 0, jnp.zeros((LANES,), jnp.float32), d3)


def _sqrt16(x):
    xi = lax.bitcast_convert_type(x, jnp.int32)
    yi = jnp.int32(0x5F3759DF) - (xi >> 1)
    y = lax.bitcast_convert_type(yi, jnp.float32)
    for _ in range(2):
        y = y * (1.5 - 0.5 * x * y * y)
    return jnp.where(x > 0, x * y, jnp.zeros((LANES,), jnp.float32))


def _splat_i32(x):
    return jnp.zeros((LANES,), jnp.int32) + x


def _body(z_hbm, b_hbm, i_hbm, j_hbm, out_hbm,
          i_all, j_all, zi0, zj0, zi1, zj1,
          b_all, acc_v, z_sp,
          si0, sj0, si1, sj1, sb):
    sid = lax.axis_index("s")
    wid = sid * NC + lax.axis_index("c")
    lane = lax.iota(jnp.int32, LANES)

    rows_per_sid = BATCH // NS
    cz = pltpu.async_copy(
        z_hbm.at[pl.ds(sid * rows_per_sid, rows_per_sid)],
        z_sp.at[pl.ds(sid * rows_per_sid, rows_per_sid)], sb)
    cb = pltpu.async_copy(b_hbm, b_all, sb)
    ci = pltpu.async_copy(i_hbm.at[pl.ds(wid * N_CHUNKS, N_CHUNKS)], i_all, si0)
    cj = pltpu.async_copy(j_hbm.at[pl.ds(wid * N_CHUNKS, N_CHUNKS)], j_all, sj0)
    ci.wait()
    cj.wait()

    def fix(g, carry):
        row = _splat_i32(g >> 3)
        col = (g & 7) * LANES + lane
        iv = plsc.load_gather(i_all, [row, col])
        jv = plsc.load_gather(j_all, [row, col])
        jf = jnp.where(iv == jv, (jv + 1) & (BATCH - 1), jv)
        plsc.store_scatter(j_all, [row, col], jf)
        return carry

    lax.fori_loop(0, N_CHUNKS * GROUPS, fix, 0)
    cz.wait()
    cb.wait()
    plsc.subcore_barrier()

    def issue(c, zi_v, zj_v, sem_i, sem_j):
        pltpu.async_copy(z_sp.at[i_all.at[c]], zi_v, sem_i)
        pltpu.async_copy(z_sp.at[j_all.at[c]], zj_v, sem_j)

    def wait(c, zi_v, zj_v, sem_i, sem_j):
        pltpu.make_async_copy(z_sp.at[i_all.at[c]], zi_v, sem_i).wait()
        pltpu.make_async_copy(z_sp.at[j_all.at[c]], zj_v, sem_j).wait()

    def compute(c, zi_v, zj_v, acc):
        def group(g, acc):
            p = g * LANES + lane
            row = _splat_i32(c)
            iv = plsc.load_gather(i_all, [row, p])
            jv = plsc.load_gather(j_all, [row, p])
            d2 = jnp.zeros((LANES,), jnp.float32)
            for d in range(DIM):
                col = (lane + d) & (DIM - 1)
                a = plsc.load_gather(zi_v, [p, col])
                b = plsc.load_gather(zj_v, [p, col])
                df = a - b
                d2 = d2 + df * df
            d_lat = _sqrt16(d2)
            bi = plsc.load_gather(b_all, [iv])
            bj = plsc.load_gather(b_all, [jv])
            err = d_lat - _padic_d3(bi, bj)
            return acc + err * err

        return lax.fori_loop(0, GROUPS, group, acc)

    issue(0, zi0, zj0, si0, sj0)

    def ring(it, acc):
        c0 = 2 * it
        issue(c0 + 1, zi1, zj1, si1, sj1)
        wait(c0, zi0, zj0, si0, sj0)
        acc = compute(c0, zi0, zj0, acc)
        issue(c0 + 2, zi0, zj0, si0, sj0)
        wait(c0 + 1, zi1, zj1, si1, sj1)
        return compute(c0 + 1, zi1, zj1, acc)

    acc = lax.fori_loop(0, N_CHUNKS // 2 - 1, ring,
                        jnp.zeros((LANES,), jnp.float32))
    issue(N_CHUNKS - 1, zi1, zj1, si1, sj1)
    wait(N_CHUNKS - 2, zi0, zj0, si0, sj0)
    acc = compute(N_CHUNKS - 2, zi0, zj0, acc)
    wait(N_CHUNKS - 1, zi1, zj1, si1, sj1)
    acc = compute(N_CHUNKS - 1, zi1, zj1, acc)

    acc_v[...] = acc * (1.0 / N_PAIRS)
    pltpu.sync_copy(acc_v, out_hbm.at[pl.ds(wid * LANES, LANES)])


_sc_call = functools.partial(
    pl.kernel,
    out_type=jax.ShapeDtypeStruct((NW * LANES,), jnp.float32),
    mesh=plsc.VectorSubcoreMesh(core_axis_name="c", subcore_axis_name="s"),
    compiler_params=pltpu.CompilerParams(
        needs_layout_passes=False, use_tc_tiling_on_sc=False),
    scratch_types=[
        pltpu.VMEM((N_CHUNKS, CHUNK), jnp.int32),
        pltpu.VMEM((N_CHUNKS, CHUNK), jnp.int32),
        pltpu.VMEM((CHUNK, DIM), jnp.float32),
        pltpu.VMEM((CHUNK, DIM), jnp.float32),
        pltpu.VMEM((CHUNK, DIM), jnp.float32),
        pltpu.VMEM((CHUNK, DIM), jnp.float32),
        pltpu.VMEM((BATCH,), jnp.int32),
        pltpu.VMEM((LANES,), jnp.float32),
        pltpu.VMEM_SHARED((BATCH, DIM), jnp.float32),
        pltpu.SemaphoreType.DMA,
        pltpu.SemaphoreType.DMA,
        pltpu.SemaphoreType.DMA,
        pltpu.SemaphoreType.DMA,
        pltpu.SemaphoreType.DMA,
    ],
)(_body)


@jax.jit
def kernel(z, batch_indices, i_idx, j_idx):
    i2 = i_idx.reshape(N_PAIRS // CHUNK, CHUNK)
    j2 = j_idx.reshape(N_PAIRS // CHUNK, CHUNK)
    partials = _sc_call(z, batch_indices, i2, j2)
    return jnp.sum(partials)
